# Initial kernel scaffold; baseline (speedup 1.0000x reference)
import jax
import jax.numpy as jnp
from jax.experimental import pallas as pl


def kernel(species, table):
    raise NotImplementedError("write your pallas kernel here")


# speedup vs baseline: 5.4312x; 5.4312x over previous
import functools

import jax
import jax.numpy as jnp
from jax import lax
from jax.experimental import pallas as pl
from jax.experimental.pallas import tpu as pltpu
from jax.experimental.pallas import tpu_sc as plsc

B_ROWS = 4096
S_COLS = 50
SP = 56
D = 110
DP = 128
V = 96

NC = 2
NS = 16
NW = NC * NS
B_PER_W = B_ROWS // NW
NB = 4
N_STEPS = B_PER_W // NB
IDX_PER_STEP = NB * SP
IDX_PER_G = IDX_PER_STEP // 2
IDX_PER_W = B_PER_W * SP


def _sc_gather(idx_pad, table_pad):
    mesh = plsc.VectorSubcoreMesh(core_axis_name="c", subcore_axis_name="s")

    @functools.partial(
        pl.kernel,
        mesh=mesh,
        out_type=jax.ShapeDtypeStruct((B_ROWS, SP, DP), jnp.float32),
        scratch_types=[
            pltpu.VMEM_SHARED((V, DP), jnp.float32),
            pltpu.VMEM((IDX_PER_W,), jnp.int32),
            pltpu.VMEM((NB * SP, DP), jnp.float32),
            pltpu.SemaphoreType.DMA,
        ],
    )
    def k(idx_hbm, tab_hbm, out_hbm, tab_sp, idx_v, rows_v, sem):
        cid = lax.axis_index("c")
        sid = lax.axis_index("s")
        wid = sid * NC + cid
        b0 = wid * B_PER_W

        @pl.when(sid == 0)
        def _():
            pltpu.sync_copy(tab_hbm, tab_sp)

        plsc.subcore_barrier()

        pltpu.sync_copy(idx_hbm.at[pl.ds(wid * IDX_PER_W, IDX_PER_W)], idx_v)
        rows_3d = rows_v.reshape(NB, SP, DP)

        def step(i, carry):
            for g in range(2):
                idx_sl = idx_v.at[pl.ds(i * IDX_PER_STEP + g * IDX_PER_G,
                                        IDX_PER_G)]
                dst = rows_v.at[pl.ds(g * IDX_PER_G, IDX_PER_G)]
                pltpu.async_copy(tab_sp.at[idx_sl], dst, sem).wait()
            pltpu.sync_copy(rows_3d, out_hbm.at[pl.ds(b0 + i * NB, NB)])
            return carry

        lax.fori_loop(0, N_STEPS, step, 0)

    return k(idx_pad, table_pad)


def kernel(species, table):
    idx_pad = jnp.pad(species, ((0, 0), (0, SP - S_COLS)), mode="edge")
    table_pad = jnp.pad(table, ((0, 0), (0, DP - D)))
    padded = _sc_gather(idx_pad.reshape(B_ROWS * SP), table_pad)
    return padded[:, :S_COLS, :D]


# speedup vs baseline: 6.4619x; 1.1898x over previous
import functools

import jax
import jax.numpy as jnp
from jax import lax
from jax.experimental import pallas as pl
from jax.experimental.pallas import tpu as pltpu
from jax.experimental.pallas import tpu_sc as plsc

B_ROWS = 4096
S_COLS = 50
SP = 56
D = 110
DP = 128
V = 96

NC = 2
NS = 16
NW = NC * NS
B_PER_W = B_ROWS // NW
NB = 8
N_STEPS = B_PER_W // NB
IDX_PER_STEP = NB * SP
IDX_PER_G = 112
N_G = IDX_PER_STEP // IDX_PER_G
IDX_PER_W = B_PER_W * SP
NBUF = 2


def _sc_gather(idx_pad, table_pad):
    mesh = plsc.VectorSubcoreMesh(core_axis_name="c", subcore_axis_name="s")

    @functools.partial(
        pl.kernel,
        mesh=mesh,
        out_type=jax.ShapeDtypeStruct((B_ROWS, SP, DP), jnp.float32),
        scratch_types=[
            pltpu.VMEM_SHARED((V, DP), jnp.float32),
            pltpu.VMEM((IDX_PER_W,), jnp.int32),
            pltpu.VMEM((NBUF * NB * SP, DP), jnp.float32),
            pltpu.SemaphoreType.DMA,
            pltpu.SemaphoreType.DMA,
        ],
    )
    def k(idx_hbm, tab_hbm, out_hbm, tab_sp, idx_v, rows_v, sem_g, sem_w):
        cid = lax.axis_index("c")
        sid = lax.axis_index("s")
        wid = sid * NC + cid
        b0 = wid * B_PER_W

        @pl.when(sid == 0)
        def _():
            pltpu.sync_copy(tab_hbm, tab_sp)

        plsc.subcore_barrier()

        pltpu.sync_copy(idx_hbm.at[pl.ds(wid * IDX_PER_W, IDX_PER_W)], idx_v)
        rows_3d = rows_v.reshape(NBUF * NB, SP, DP)

        def step(i, carry):
            buf = i & 1

            @pl.when(i >= NBUF)
            def _():
                pltpu.make_async_copy(
                    rows_3d.at[pl.ds(0, NB)],
                    out_hbm.at[pl.ds(b0, NB)],
                    sem_w,
                ).wait()

            copies = []
            for g in range(N_G):
                idx_sl = idx_v.at[pl.ds(i * IDX_PER_STEP + g * IDX_PER_G,
                                        IDX_PER_G)]
                dst = rows_v.at[pl.ds(buf * NB * SP + g * IDX_PER_G,
                                      IDX_PER_G)]
                copies.append(pltpu.async_copy(tab_sp.at[idx_sl], dst, sem_g))
            for c in copies:
                c.wait()
            pltpu.async_copy(
                rows_3d.at[pl.ds(buf * NB, NB)],
                out_hbm.at[pl.ds(b0 + i * NB, NB)],
                sem_w,
            )
            return carry

        lax.fori_loop(0, N_STEPS, step, 0)

        for _ in range(NBUF):
            pltpu.make_async_copy(
                rows_3d.at[pl.ds(0, NB)],
                out_hbm.at[pl.ds(b0, NB)],
                sem_w,
            ).wait()

    return k(idx_pad, table_pad)


def kernel(species, table):
    idx_pad = jnp.pad(species, ((0, 0), (0, SP - S_COLS)), mode="edge")
    table_pad = jnp.pad(table, ((0, 0), (0, DP - D)))
    padded = _sc_gather(idx_pad.reshape(B_ROWS * SP), table_pad)
    return padded[:, :S_COLS, :D]
